# baseline (speedup 1.0000x reference)
import functools
import math

import jax
import jax.numpy as jnp
from jax import lax
from jax.experimental import pallas as pl
from jax.experimental.pallas import tpu as pltpu
from jax.experimental.pallas import tpu_sc as plsc

B = 8
SEQ = 45
F_IN = 30
HID = 256
DMODEL = 512
NDB = 4096
DFLAT = SEQ * DMODEL
NH = 8
DK = 64
LT = 32
VOCAB = 556
FF = 2048
CHUNK = 128
NSTEP = NDB // CHUNK


def _mmT(a, b):
    return lax.dot_general(a, b, (((1,), (1,)), ((), ())),
                           preferred_element_type=jnp.float32)


def _mm(a, b):
    return lax.dot_general(a, b, (((1,), (0,)), ((), ())),
                           preferred_element_type=jnp.float32)


def _enc_body(x_ref, wlin_ref, wih_f_ref, whh_f_ref, bih_f_ref, bhh_f_ref,
              wih_r_ref, whh_r_ref, bih_r_ref, bhh_r_ref, out_ref,
              gf_ref, gr_ref):
    a = _mmT(x_ref[...], wlin_ref[...])
    wih_f = wih_f_ref[...]
    wih_r = wih_r_ref[...]
    for b in range(B):
        ab = a[b * F_IN:(b + 1) * F_IN, :]
        gf_ref[:, b, :] = lax.dot_general(
            ab, wih_f, (((0,), (1,)), ((), ())),
            preferred_element_type=jnp.float32)
        gr_ref[:, b, :] = lax.dot_general(
            ab, wih_r, (((0,), (1,)), ((), ())),
            preferred_element_type=jnp.float32)

    bias_f = bih_f_ref[...] + bhh_f_ref[...]
    bias_r = bih_r_ref[...] + bhh_r_ref[...]
    whh_f = whh_f_ref[...]
    whh_r = whh_r_ref[...]

    def step(t, carry):
        hf, cf, hr, cr = carry
        gf = gf_ref[t] + _mmT(hf, whh_f) + bias_f
        gr = gr_ref[SEQ - 1 - t] + _mmT(hr, whh_r) + bias_r
        i_f = jax.nn.sigmoid(gf[:, 0:HID])
        f_f = jax.nn.sigmoid(gf[:, HID:2 * HID])
        g_f = jnp.tanh(gf[:, 2 * HID:3 * HID])
        o_f = jax.nn.sigmoid(gf[:, 3 * HID:4 * HID])
        cf = f_f * cf + i_f * g_f
        hf = o_f * jnp.tanh(cf)
        i_r = jax.nn.sigmoid(gr[:, 0:HID])
        f_r = jax.nn.sigmoid(gr[:, HID:2 * HID])
        g_r = jnp.tanh(gr[:, 2 * HID:3 * HID])
        o_r = jax.nn.sigmoid(gr[:, 3 * HID:4 * HID])
        cr = f_r * cr + i_r * g_r
        hr = o_r * jnp.tanh(cr)
        out_ref[:, pl.ds(t, 1), 0:HID] = hf[:, None, :]
        out_ref[:, pl.ds(SEQ - 1 - t, 1), HID:2 * HID] = hr[:, None, :]
        return hf, cf, hr, cr

    z = jnp.zeros((B, HID), jnp.float32)
    lax.fori_loop(0, SEQ, step, (z, z, z, z))


def _encoder(x, p):
    return pl.pallas_call(
        _enc_body,
        out_shape=jax.ShapeDtypeStruct((B, SEQ, DMODEL), jnp.float32),
        scratch_shapes=[pltpu.VMEM((SEQ, B, 4 * HID), jnp.float32),
                        pltpu.VMEM((SEQ, B, 4 * HID), jnp.float32)],
    )(x, p['w_lin1'], p['wih_f'], p['whh_f'], p['bih_f'], p['bhh_f'],
      p['wih_r'], p['whh_r'], p['bih_r'], p['bhh_r'])


def _dbattn_body(q_ref, db_ref, o_ref, acc_ref, m_ref, l_ref):
    i = pl.program_id(0)

    @pl.when(i == 0)
    def _():
        acc_ref[...] = jnp.zeros_like(acc_ref)
        m_ref[...] = jnp.full_like(m_ref, -1e30)
        l_ref[...] = jnp.zeros_like(l_ref)

    db = db_ref[...]
    s = _mmT(q_ref[...], db) * (1.0 / math.sqrt(512.0))
    m_prev = m_ref[...]
    m_new = jnp.maximum(m_prev, jnp.max(s, axis=1, keepdims=True))
    alpha = jnp.exp(m_prev - m_new)
    pmat = jnp.exp(s - m_new)
    l_ref[...] = l_ref[...] * alpha + jnp.sum(pmat, axis=1, keepdims=True)
    m_ref[...] = m_new
    acc_ref[...] = acc_ref[...] * alpha[:, 0:1] + _mm(pmat, db)

    @pl.when(i == NSTEP - 1)
    def _():
        o_ref[...] = acc_ref[...] / l_ref[:, 0:1]


def _db_attn(q, kdb):
    return pl.pallas_call(
        _dbattn_body,
        grid=(NSTEP,),
        in_specs=[pl.BlockSpec((B, DFLAT), lambda i: (0, 0)),
                  pl.BlockSpec((CHUNK, DFLAT), lambda i: (i, 0))],
        out_specs=pl.BlockSpec((B, DFLAT), lambda i: (0, 0)),
        out_shape=jax.ShapeDtypeStruct((B, DFLAT), jnp.float32),
        scratch_shapes=[pltpu.VMEM((B, DFLAT), jnp.float32),
                        pltpu.VMEM((B, CHUNK), jnp.float32),
                        pltpu.VMEM((B, CHUNK), jnp.float32)],
    )(q, kdb)


def _sims_body(e_ref, lab_ref, idx_ref, sims_ref):
    i = pl.program_id(0)
    ch = lab_ref[...]
    ev = e_ref[...]
    qinv = 1.0 / jnp.maximum(
        jnp.sqrt(jnp.sum(ev * ev, axis=1, keepdims=True)), 1e-8)
    ninv = 1.0 / jnp.maximum(
        jnp.sqrt(jnp.sum(ch * ch, axis=1, keepdims=True)), 1e-8)
    s = _mmT(ev * qinv, ch * ninv)
    sims_ref[:, pl.ds(i * CHUNK, CHUNK)] = s

    @pl.when(i == NSTEP - 1)
    def _():
        sims = sims_ref[...]
        colidx = lax.broadcasted_iota(jnp.int32, (B, NDB), 1)
        work = sims
        blocks = []
        sub_iota = lax.broadcasted_iota(jnp.int32, (B, 48), 1)
        for _k in range(3):
            mx = jnp.max(work, axis=1, keepdims=True)
            cand = jnp.where(work == mx, colidx, jnp.int32(2147483647))
            a = jnp.min(cand, axis=1, keepdims=True)
            blocks.append(jnp.minimum(a * SEQ + sub_iota, NDB * SEQ - 1))
            work = jnp.where(colidx == a, jnp.float32(-3e38), work)
        idx_ref[...] = jnp.concatenate(blocks, axis=1)


def _sims_topk(e_flat, lab):
    return pl.pallas_call(
        _sims_body,
        grid=(NSTEP,),
        in_specs=[pl.BlockSpec((B, DFLAT), lambda i: (0, 0)),
                  pl.BlockSpec((CHUNK, DFLAT), lambda i: (i, 0))],
        out_specs=pl.BlockSpec((B, 144), lambda i: (0, 0)),
        out_shape=jax.ShapeDtypeStruct((B, 144), jnp.int32),
        scratch_shapes=[pltpu.VMEM((B, NDB), jnp.float32)],
    )(e_flat, lab)


def _sc_gather(lab2, sub_idx):
    mesh = plsc.VectorSubcoreMesh(core_axis_name="c", subcore_axis_name="s")

    @functools.partial(
        pl.kernel, mesh=mesh,
        out_type=jax.ShapeDtypeStruct((24, 48, DMODEL), jnp.float32),
        scratch_types=[pltpu.VMEM((48,), jnp.int32),
                       pltpu.VMEM((48, DMODEL), jnp.float32),
                       pltpu.SemaphoreType.DMA],
    )
    def k(lab_hbm, idx_hbm, out_hbm, sub_v, rows_v, sem):
        info = plsc.get_sparse_core_info()
        wid = lax.axis_index("s") * info.num_cores + lax.axis_index("c")

        @pl.when(wid < 24)
        def _():
            pltpu.sync_copy(idx_hbm.at[wid], sub_v)
            pltpu.async_copy(lab_hbm.at[sub_v], rows_v, sem).wait()
            pltpu.sync_copy(rows_v, out_hbm.at[wid])

    return k(lab2, sub_idx)


def _ln(x, g, b):
    m = jnp.mean(x, axis=-1, keepdims=True)
    d = x - m
    v = jnp.mean(d * d, axis=-1, keepdims=True)
    return d / jnp.sqrt(v + 1e-5) * g + b


def _heads_attn(q, k, v, nkv, masked):
    q3 = q.reshape(B, LT, DMODEL)
    k3 = k.reshape(B, nkv, DMODEL)
    v3 = v.reshape(B, nkv, DMODEL)
    outs = []
    for h in range(NH):
        sl = slice(h * DK, (h + 1) * DK)
        qh = q3[:, :, sl]
        kh = k3[:, :, sl]
        vh = v3[:, :, sl]
        s = lax.dot_general(qh, kh, (((2,), (2,)), ((0,), (0,))),
                            preferred_element_type=jnp.float32) / 8.0
        if masked:
            ri = lax.broadcasted_iota(jnp.int32, (B, LT, nkv), 1)
            ci = lax.broadcasted_iota(jnp.int32, (B, LT, nkv), 2)
            s = jnp.where(ci <= ri, s, jnp.float32(-1e20))
        m = jnp.max(s, axis=-1, keepdims=True)
        e = jnp.exp(s - m)
        p = e / jnp.sum(e, axis=-1, keepdims=True)
        o = lax.dot_general(p, vh, (((2,), (1,)), ((0,), (0,))),
                            preferred_element_type=jnp.float32)
        outs.append(o)
    return jnp.concatenate(outs, axis=2).reshape(B * LT, DMODEL)


def _dec_self_body(tgt_ref, emb_ref, *refs):
    out_ref = refs[-1]
    wr = refs[:-1]
    tcol = tgt_ref[...]
    oh = (tcol == lax.broadcasted_iota(jnp.int32, (B * LT, VOCAB), 1))
    we = _mm(oh.astype(jnp.float32), emb_ref[...])
    v = we
    for l in range(2):
        (wq, bq, wk, bk, wv, bv, wp, bp,
         l1g, l1b, l2g, l2b, f1, fb1, f2, fb2) = wr[l * 16:(l + 1) * 16]
        q = _mmT(we, wq[...]) + bq[...]
        k = _mmT(we, wk[...]) + bk[...]
        vv = _mmT(v, wv[...]) + bv[...]
        o = _heads_attn(q, k, vv, LT, True)
        a = _mmT(o, wp[...]) + bp[...]
        x2 = _ln(a + we, l1g[...], l1b[...])
        h1 = _mmT(x2, f1[...]) + fb1[...]
        h1 = 0.5 * h1 * (1.0 + lax.erf(h1 * (1.0 / math.sqrt(2.0))))
        ff = _mmT(h1, f2[...]) + fb2[...]
        v = _ln(x2 + ff, l2g[...], l2b[...])
    out_ref[...] = v


def _dec_self(tgt_col, p):
    args = [tgt_col, p['emb']]
    for l in range(2):
        pre = 'enc%d_' % l
        for nm in ('wq', 'bq', 'wk', 'bk', 'wv', 'bv', 'wp', 'bp',
                   'ln1g', 'ln1b', 'ln2g', 'ln2b',
                   'ffw1', 'ffb1', 'ffw2', 'ffb2'):
            args.append(p[pre + nm])
    return pl.pallas_call(
        _dec_self_body,
        out_shape=jax.ShapeDtypeStruct((B * LT, DMODEL), jnp.float32),
    )(*args)


def _dec_cross_body(v_ref, e2_ref, lab_ref, wq_ref, bq_ref, wk_ref, bk_ref,
                    wv_ref, bv_ref, wp_ref, bp_ref, wpr_ref, bpr_ref,
                    out_ref):
    inp = jnp.concatenate([e2_ref[...], lab_ref[...]], axis=1)
    inp2 = inp.reshape(B * 180, DMODEL)
    vdec = v_ref[...]
    q = _mmT(vdec, wq_ref[...]) + bq_ref[...]
    k = _mmT(inp2, wk_ref[...]) + bk_ref[...]
    vv = _mmT(inp2, wv_ref[...]) + bv_ref[...]
    o = _heads_attn(q, k, vv, 180, False)
    a = _mmT(o, wp_ref[...]) + bp_ref[...]
    out_ref[...] = _mmT(a, wpr_ref[...]) + bpr_ref[...]


def _dec_cross(vdec, e2, labels, p):
    return pl.pallas_call(
        _dec_cross_body,
        out_shape=jax.ShapeDtypeStruct((B * LT, VOCAB), jnp.float32),
    )(vdec, e2, labels, p['dec_wq'], p['dec_bq'], p['dec_wk'], p['dec_bk'],
      p['dec_wv'], p['dec_bv'], p['dec_wp'], p['dec_bp'],
      p['w_proj'], p['b_proj'])


def kernel(src, data, label, tgt, params):
    p = params
    x = src.transpose(0, 2, 1, 3).reshape(B, 1500, F_IN)
    x = jnp.swapaxes(x, 1, 2).reshape(B * F_IN, 1500)
    e_imu = _encoder(x, p)
    qflat = e_imu.reshape(B, DFLAT)
    e_flat = _db_attn(qflat, data.reshape(NDB, DFLAT))
    sub = _sims_topk(e_flat, label.reshape(NDB, DFLAT))
    gathered = _sc_gather(label.reshape(NDB * SEQ, DMODEL),
                          sub.reshape(24, 48))
    labels = gathered[:, :SEQ, :].reshape(B, 3 * SEQ, DMODEL)
    vdec = _dec_self(tgt.reshape(B * LT, 1), p)
    out = _dec_cross(vdec, e_flat.reshape(B, SEQ, DMODEL), labels, p)
    return out.reshape(B, LT, VOCAB)
